# baseline (device time: 76432 ns/iter reference)
import jax
import jax.numpy as jnp
from jax import lax
from jax.experimental import pallas as pl
from jax.experimental.pallas import tpu as pltpu

T = 2048
D = 4096
V_SHARD = 8192
V_SUB = 2048
VT = 512
NT = V_SUB // VT

CH = 256
NCH = T // CH

N_PEERS = 7


def kernel(x, W, labels):
    labels_col = labels.reshape(T, 1)
    q = 2 * lax.axis_index("x") + lax.axis_index("z")
    q_arr = jnp.asarray(q, jnp.int32).reshape(1)

    def body(q_ref, x_hbm, w_ref, lab_ref, out_ref,
             xbf_ref, xst_ref, lbuf_ref, s_ref, ll_ref,
             send_ref, recv_ref,
             x_sems, send_sems, recv_sems):
        v = pl.program_id(0)
        my_x = lax.axis_index("x")
        my_y = lax.axis_index("y")
        my_z = lax.axis_index("z")
        ones_col = jnp.ones((VT, 1), jnp.float32)

        def process(slot, vv):
            logits = lbuf_ref[slot]
            e = jnp.exp(logits)
            s_ref[...] += jnp.dot(e, ones_col,
                                  preferred_element_type=jnp.float32)
            v0 = my_y * V_SHARD + q_ref[0] * V_SUB + vv * VT
            ids = lax.broadcasted_iota(jnp.int32, (T, VT), 1) + v0
            sel = jnp.where(ids == lab_ref[...], logits, 0.0)
            ll_ref[...] += jnp.dot(sel, ones_col,
                                   preferred_element_type=jnp.float32)

        @pl.when(v == 0)
        def _load_x():
            s_ref[...] = jnp.zeros_like(s_ref)
            ll_ref[...] = jnp.zeros_like(ll_ref)
            copies = [
                pltpu.make_async_copy(
                    x_hbm.at[pl.ds(c * CH, CH), :],
                    xst_ref.at[c % 2],
                    x_sems.at[c % 2],
                )
                for c in range(NCH)
            ]
            copies[0].start()
            for c in range(NCH):
                if c + 1 < NCH:
                    copies[c + 1].start()
                copies[c].wait()
                xbf_ref[pl.ds(c * CH, CH), :] = (
                    xst_ref[c % 2].astype(jnp.bfloat16))

        wbf = w_ref[...].astype(jnp.bfloat16)
        lbuf_ref[v % 2] = jnp.dot(xbf_ref[...], wbf,
                                  preferred_element_type=jnp.float32)

        @pl.when(v > 0)
        def _process_prev():
            process((v - 1) % 2, v - 1)

        @pl.when(v == NT - 1)
        def _finish():
            process((NT - 1) % 2, NT - 1)

            acc = jnp.concatenate(
                [s_ref[...].reshape(1, T), ll_ref[...].reshape(1, T)],
                axis=0)
            send_ref[...] = acc
            rdmas = []
            for m in range(1, 8):
                bx, by, bz = (m >> 2) & 1, (m >> 1) & 1, m & 1
                px = 1 - my_x if bx else my_x
                py = 1 - my_y if by else my_y
                pz = 1 - my_z if bz else my_z
                r = pltpu.make_async_remote_copy(
                    src_ref=send_ref,
                    dst_ref=recv_ref.at[m - 1],
                    send_sem=send_sems.at[m - 1],
                    recv_sem=recv_sems.at[m - 1],
                    device_id=(px, py, pz),
                    device_id_type=pl.DeviceIdType.MESH,
                )
                r.start()
                rdmas.append(r)
            for r in rdmas:
                r.wait()
            tot = acc
            for m in range(1, 8):
                tot = tot + recv_ref[m - 1]
            out_ref[...] = jnp.log(tot[0:1, :]) - tot[1:2, :]

    grid_spec = pltpu.PrefetchScalarGridSpec(
        num_scalar_prefetch=1,
        grid=(NT,),
        in_specs=[
            pl.BlockSpec(memory_space=pl.ANY),
            pl.BlockSpec((D, VT), lambda v, q: (0, q[0] * NT + v)),
            pl.BlockSpec((T, 1), lambda v, q: (0, 0)),
        ],
        out_specs=pl.BlockSpec((1, T), lambda v, q: (0, 0)),
        scratch_shapes=[
            pltpu.VMEM((T, D), jnp.bfloat16),
            pltpu.VMEM((2, CH, D), jnp.float32),
            pltpu.VMEM((2, T, VT), jnp.float32),
            pltpu.VMEM((T, 1), jnp.float32),
            pltpu.VMEM((T, 1), jnp.float32),
            pltpu.VMEM((2, T), jnp.float32),
            pltpu.VMEM((N_PEERS, 2, T), jnp.float32),
            pltpu.SemaphoreType.DMA((2,)),
            pltpu.SemaphoreType.DMA((N_PEERS,)),
            pltpu.SemaphoreType.DMA((N_PEERS,)),
        ],
    )

    out = pl.pallas_call(
        body,
        grid_spec=grid_spec,
        out_shape=jax.ShapeDtypeStruct((1, T), jnp.float32),
        compiler_params=pltpu.CompilerParams(
            vmem_limit_bytes=60 * 1024 * 1024),
    )(q_arr, x, W, labels_col)
    return out.reshape(T)
